# baseline (speedup 1.0000x reference)
import functools

import jax
import jax.numpy as jnp
from jax import lax
from jax.experimental import pallas as pl
from jax.experimental.pallas import tpu as pltpu
from jax.experimental.pallas import tpu_sc as plsc

_NC = 2
_NS = 16
_NW = _NC * _NS
_CH = 128
_EMB = 8


def _sc_gather(user_idx, item_idx, user_table, item_table):
    B = user_idx.shape[0]
    bpw = B // _NW
    nch = bpw // _CH

    mesh = plsc.VectorSubcoreMesh(core_axis_name="c", subcore_axis_name="s")

    @functools.partial(
        pl.kernel,
        out_type=(
            jax.ShapeDtypeStruct((B, _EMB), jnp.float32),
            jax.ShapeDtypeStruct((B, _EMB), jnp.float32),
        ),
        mesh=mesh,
        scratch_types=[
            pltpu.VMEM((nch, _CH), jnp.int32),
            pltpu.VMEM((nch, _CH), jnp.int32),
            pltpu.VMEM((nch, _CH, _EMB), jnp.float32),
            pltpu.VMEM((nch, _CH, _EMB), jnp.float32),
            pltpu.SemaphoreType.DMA,
        ],
        compiler_params=pltpu.CompilerParams(use_tc_tiling_on_sc=False),
    )
    def gather(uidx_h, iidx_h, utab_h, itab_h, uout_h, iout_h,
               uidx_v, iidx_v, urow_v, irow_v, sem):
        wid = lax.axis_index("s") * _NC + lax.axis_index("c")
        base = wid * bpw
        for j in range(nch):
            pltpu.sync_copy(uidx_h.at[pl.ds(base + j * _CH, _CH)], uidx_v.at[j])
            pltpu.sync_copy(iidx_h.at[pl.ds(base + j * _CH, _CH)], iidx_v.at[j])
        copies = []
        for j in range(nch):
            copies.append(pltpu.async_copy(utab_h.at[uidx_v.at[j]], urow_v.at[j], sem))
            copies.append(pltpu.async_copy(itab_h.at[iidx_v.at[j]], irow_v.at[j], sem))
        for c in copies:
            c.wait()
        for j in range(nch):
            pltpu.sync_copy(urow_v.at[j], uout_h.at[pl.ds(base + j * _CH, _CH)])
            pltpu.sync_copy(irow_v.at[j], iout_h.at[pl.ds(base + j * _CH, _CH)])

    return gather(user_idx, item_idx, user_table, item_table)


def _tc_dense(u, i, feat, Wf, bf, W1u, W1i, W1f, b1):
    B = feat.shape[0]
    BB = 2048

    def body(u_ref, i_ref, f_ref, wf_ref, bf_ref, w1u_ref, w1i_ref, w1f_ref,
             b1_ref, o_ref):
        dn = (((1,), (1,)), ((), ()))
        fenc = lax.dot_general(f_ref[...], wf_ref[...], dn,
                               preferred_element_type=jnp.float32) + bf_ref[...]
        h = (lax.dot_general(u_ref[...], w1u_ref[...], dn,
                             preferred_element_type=jnp.float32)
             + lax.dot_general(i_ref[...], w1i_ref[...], dn,
                               preferred_element_type=jnp.float32)
             + lax.dot_general(fenc, w1f_ref[...], dn,
                               preferred_element_type=jnp.float32)
             + b1_ref[...])
        o_ref[...] = jnp.maximum(h, 0.0)

    return pl.pallas_call(
        body,
        grid=(B // BB,),
        in_specs=[
            pl.BlockSpec((BB, _EMB), lambda g: (g, 0)),
            pl.BlockSpec((BB, _EMB), lambda g: (g, 0)),
            pl.BlockSpec((BB, 128), lambda g: (g, 0)),
            pl.BlockSpec((_EMB, 128), lambda g: (0, 0)),
            pl.BlockSpec((1, _EMB), lambda g: (0, 0)),
            pl.BlockSpec((64, _EMB), lambda g: (0, 0)),
            pl.BlockSpec((64, _EMB), lambda g: (0, 0)),
            pl.BlockSpec((64, _EMB), lambda g: (0, 0)),
            pl.BlockSpec((1, 64), lambda g: (0, 0)),
        ],
        out_specs=pl.BlockSpec((BB, 64), lambda g: (g, 0)),
        out_shape=jax.ShapeDtypeStruct((B, 64), jnp.float32),
    )(u, i, feat, Wf, bf, W1u, W1i, W1f, b1)


def kernel(user_idx, item_idx, feat_vecs, user_table, item_table, Wf, bf, W1, b1):
    u, i = _sc_gather(user_idx.astype(jnp.int32), item_idx.astype(jnp.int32),
                      user_table, item_table)
    W1u = W1[:, 0:_EMB]
    W1i = W1[:, _EMB:2 * _EMB]
    W1f = W1[:, 2 * _EMB:3 * _EMB]
    return _tc_dense(u, i, feat_vecs, Wf, bf.reshape(1, _EMB),
                     W1u, W1i, W1f, b1.reshape(1, 64))
